# Initial kernel scaffold; baseline (speedup 1.0000x reference)
import jax
import jax.numpy as jnp
from jax.experimental import pallas as pl


def kernel(x):
    raise NotImplementedError("write your pallas kernel here")


# speedup vs baseline: 10.8936x; 10.8936x over previous
import functools

import jax
import jax.numpy as jnp
from jax import lax
from jax.experimental import pallas as pl
from jax.experimental.pallas import tpu as pltpu
from jax.experimental.pallas import tpu_sc as plsc

N = 320000
D = 128
NC = 2
NS = 16
NW = NC * NS
ROWS_PER_W = N // NW
C = 200
NBUF = 2
NCHUNK = ROWS_PER_W // C
NGROUP = NCHUNK // NBUF
LANES = 16
JL = D // LANES


def _sc_partial_sums(x_hbm, out_hbm, buf0, buf1, acc_v, sem0, sem1):
    wid = lax.axis_index("s") * NC + lax.axis_index("c")
    base = wid * ROWS_PER_W
    bufs = (buf0, buf1)
    sems = (sem0, sem1)

    for b in range(NBUF):
        pltpu.async_copy(x_hbm.at[pl.ds(base + b * C, C)], bufs[b], sems[b])

    def group_body(g, accs):
        for b in range(NBUF):
            chunk = g * NBUF + b
            pltpu.make_async_copy(
                x_hbm.at[pl.ds(base, C)], bufs[b], sems[b]
            ).wait()

            def row_body(r, a):
                return tuple(
                    a[j] + bufs[b][r, pl.ds(j * LANES, LANES)]
                    for j in range(JL)
                )

            accs = lax.fori_loop(0, C, row_body, accs, unroll=2)

            nxt = chunk + NBUF

            @pl.when(nxt < NCHUNK)
            def _():
                pltpu.async_copy(
                    x_hbm.at[pl.ds(base + nxt * C, C)], bufs[b], sems[b]
                )

        return accs

    zeros = tuple(jnp.zeros((LANES,), jnp.float32) for _ in range(JL))
    accs = lax.fori_loop(0, NGROUP, group_body, zeros)

    for j in range(JL):
        acc_v[0, pl.ds(j * LANES, LANES)] = accs[j]
    pltpu.sync_copy(acc_v, out_hbm.at[wid])


@functools.partial(
    pl.kernel,
    out_type=jax.ShapeDtypeStruct((NW, 1, D), jnp.float32),
    mesh=plsc.VectorSubcoreMesh(core_axis_name="c", subcore_axis_name="s"),
    scratch_types=[
        pltpu.VMEM((C, D), jnp.float32),
        pltpu.VMEM((C, D), jnp.float32),
        pltpu.VMEM((1, D), jnp.float32),
        pltpu.SemaphoreType.DMA,
        pltpu.SemaphoreType.DMA,
    ],
)
def _sc_sum_kernel(x_hbm, out_hbm, buf0, buf1, acc_v, sem0, sem1):
    _sc_partial_sums(x_hbm, out_hbm, buf0, buf1, acc_v, sem0, sem1)


def _combine_body(p_ref, o_ref):
    o_ref[...] = jnp.sum(p_ref[...], axis=0)


_combine = pl.pallas_call(
    _combine_body,
    out_shape=jax.ShapeDtypeStruct((1, D), jnp.float32),
)


def kernel(x):
    partials = _sc_sum_kernel(x)
    return _combine(partials)
